# Initial kernel scaffold; baseline (speedup 1.0000x reference)
import jax
import jax.numpy as jnp
from jax.experimental import pallas as pl


def kernel(input_ids, token_table, position_table):
    raise NotImplementedError("write your pallas kernel here")


# speedup vs baseline: 1.2995x; 1.2995x over previous
import functools

import jax
import jax.numpy as jnp
from jax import lax
from jax.experimental import pallas as pl
from jax.experimental.pallas import tpu as pltpu
from jax.experimental.pallas import tpu_sc as plsc

D = 32
L = 200
B = 4096
NC, NS = 2, 16
NW = NC * NS
ROWS = B * L
RPW = ROWS // NW
CHUNK = 1600
NIT = RPW // CHUNK
UNROLL = 8
_STREAMS = [(j * 128, 128) for j in range(12)] + [(1536, 64)]

_mesh = plsc.VectorSubcoreMesh(core_axis_name="c", subcore_axis_name="s")


@functools.partial(
    pl.kernel,
    out_type=jax.ShapeDtypeStruct((ROWS, D), jnp.float32),
    mesh=_mesh,
    scratch_types=[
        pltpu.VMEM((CHUNK,), jnp.int32),
        pltpu.VMEM((CHUNK, D), jnp.float32),
        pltpu.VMEM((L, D), jnp.float32),
        pltpu.SemaphoreType.DMA,
    ],
    compiler_params=pltpu.CompilerParams(use_tc_tiling_on_sc=False),
)
def _emb_lookup(ids_hbm, tok_hbm, pos_hbm, out_hbm, idx_v, rows_v, pos_v, sem):
    wid = lax.axis_index("s") * NC + lax.axis_index("c")
    pltpu.sync_copy(pos_hbm, pos_v)

    def chunk_body(it, carry):
        pltpu.sync_copy(ids_hbm.at[wid * NIT + it], idx_v)
        copies = [
            pltpu.async_copy(
                tok_hbm.at[idx_v.at[pl.ds(off, sz)]],
                rows_v.at[pl.ds(off, sz)],
                sem,
            )
            for off, sz in _STREAMS
        ]
        for cp in copies:
            cp.wait()

        def group_body(g, p0):
            r0 = g * UNROLL
            for k in range(UNROLL):
                r = r0 + k
                p = p0 + k
                lo = rows_v[r, pl.ds(0, 16)] + pos_v[p, pl.ds(0, 16)]
                hi = rows_v[r, pl.ds(16, 16)] + pos_v[p, pl.ds(16, 16)]
                rows_v[r, pl.ds(0, 16)] = lo
                rows_v[r, pl.ds(16, 16)] = hi
            p0 = p0 + UNROLL
            return jnp.where(p0 >= L, 0, p0)

        lax.fori_loop(0, CHUNK // UNROLL, group_body, 0)
        pltpu.sync_copy(rows_v, out_hbm.at[pl.ds(wid * RPW + it * CHUNK, CHUNK)])
        return carry

    lax.fori_loop(0, NIT, chunk_body, 0)


def kernel(input_ids, token_table, position_table):
    ids = input_ids.astype(jnp.int32).reshape(NW * NIT, CHUNK)
    out = _emb_lookup(ids, token_table, position_table)
    return out.reshape(B, L, D)


# speedup vs baseline: 1.4274x; 1.0985x over previous
import functools

import jax
import jax.numpy as jnp
from jax import lax
from jax.experimental import pallas as pl
from jax.experimental.pallas import tpu as pltpu
from jax.experimental.pallas import tpu_sc as plsc

D = 32
L = 200
B = 4096
NC, NS = 2, 16
NW = NC * NS
BPW = B // NW
CB = 8
NIT = BPW // CB
UNROLL = 8
_SPLITS = [(0, 128), (128, 72)]

_mesh = plsc.VectorSubcoreMesh(core_axis_name="c", subcore_axis_name="s")


@functools.partial(
    pl.kernel,
    out_type=jax.ShapeDtypeStruct((B, L, D), jnp.float32),
    mesh=_mesh,
    scratch_types=[
        pltpu.VMEM((CB, L), jnp.int32),
        pltpu.VMEM((CB, L, D), jnp.float32),
        pltpu.VMEM((L, D), jnp.float32),
        pltpu.SemaphoreType.DMA,
    ],
    compiler_params=pltpu.CompilerParams(use_tc_tiling_on_sc=False),
)
def _emb_lookup(ids_hbm, tok_hbm, pos_hbm, out_hbm, idx_v, rows_v, pos_v, sem):
    wid = lax.axis_index("s") * NC + lax.axis_index("c")
    pltpu.sync_copy(pos_hbm, pos_v)

    def chunk_body(it, carry):
        b0 = wid * BPW + it * CB
        pltpu.sync_copy(ids_hbm.at[pl.ds(b0, CB)], idx_v)
        copies = [
            pltpu.async_copy(
                tok_hbm.at[idx_v.at[k, pl.ds(off, sz)]],
                rows_v.at[k, pl.ds(off, sz)],
                sem,
            )
            for k in range(CB)
            for off, sz in _SPLITS
        ]
        for cp in copies:
            cp.wait()

        def group_body(g, carry2):
            l0 = g * UNROLL
            for u in range(UNROLL):
                l = l0 + u
                plo = pos_v[l, pl.ds(0, 16)]
                phi = pos_v[l, pl.ds(16, 16)]
                for k in range(CB):
                    rows_v[k, l, pl.ds(0, 16)] = rows_v[k, l, pl.ds(0, 16)] + plo
                    rows_v[k, l, pl.ds(16, 16)] = rows_v[k, l, pl.ds(16, 16)] + phi
            return carry2

        lax.fori_loop(0, L // UNROLL, group_body, 0)
        pltpu.sync_copy(rows_v, out_hbm.at[pl.ds(b0, CB)])
        return carry

    lax.fori_loop(0, NIT, chunk_body, 0)


def kernel(input_ids, token_table, position_table):
    return _emb_lookup(input_ids.astype(jnp.int32), token_table, position_table)
